# baseline (speedup 1.0000x reference)
import functools

import jax
import jax.numpy as jnp
from jax import lax
from jax.experimental import pallas as pl
from jax.experimental.pallas import tpu as pltpu
from jax.experimental.pallas import tpu_sc as plsc

K = 8192
C = 21
R = 24
NBINS = R * K
NTOPK = 64
EPS = 1e-6
NTOK = 64 * 32 * 32
NCORES = 1
NSUB = 16
NW = NCORES * NSUB
TPW = NTOK // NW
NCHUNK = TPW // 16
ZPW = NBINS // NSUB


def _sc_hist_body(idx_hbm, msk_hbm, out_hbm,
                  idx_v, msk_v, bins_v, ones_v, zeros_v, table_sh,
                  sem_ld, sem_z, sem_sc):
    c = lax.axis_index("c")
    s = lax.axis_index("s")
    wid = c * NSUB + s
    base = wid * TPW
    ld_idx = pltpu.async_copy(idx_hbm.at[pl.ds(base, TPW)], idx_v, sem_ld)
    ld_msk = pltpu.async_copy(msk_hbm.at[pl.ds(base, TPW)], msk_v, sem_ld)
    for i in range(8):
        ones_v[pl.ds(i * 16, 16)] = jnp.ones((16,), jnp.float32)
    for i in range(TPW // 16):
        zeros_v[pl.ds(i * 16, 16)] = jnp.zeros((16,), jnp.float32)
    zcopies = [
        pltpu.async_copy(zeros_v, table_sh.at[pl.ds(s * ZPW + i * TPW, TPW)],
                         sem_z)
        for i in range(ZPW // TPW)
    ]
    ld_idx.wait()
    ld_msk.wait()
    for t in range(NCHUNK):
        mi = msk_v[pl.ds(t * 16, 16)]
        ii = idx_v[pl.ds(t * 16, 16)]
        cls = jnp.minimum(mi, C)
        bins_v[t // 8, pl.ds((t % 8) * 16, 16)] = cls * K + ii
    for cp in zcopies:
        cp.wait()
    plsc.subcore_barrier()
    scopies = [
        pltpu.async_copy(ones_v, table_sh.at[bins_v.at[j]], sem_sc, add=True)
        for j in range(NCHUNK // 8)
    ]
    for cp in scopies:
        cp.wait()
    plsc.subcore_barrier()
    pltpu.sync_copy(table_sh.at[pl.ds(s * ZPW, ZPW)],
                    out_hbm.at[c, pl.ds(s * ZPW, ZPW)])


@functools.cache
def _make_sc_hist():
    return pl.kernel(
        _sc_hist_body,
        out_type=jax.ShapeDtypeStruct((NCORES, NBINS), jnp.float32),
        mesh=plsc.VectorSubcoreMesh(core_axis_name="c", subcore_axis_name="s",
                                    num_cores=NCORES, num_subcores=NSUB),
        scratch_types=[
            pltpu.VMEM((TPW,), jnp.int32),
            pltpu.VMEM((TPW,), jnp.int32),
            pltpu.VMEM((NCHUNK // 8, 128), jnp.int32),
            pltpu.VMEM((128,), jnp.float32),
            pltpu.VMEM((TPW,), jnp.float32),
            pltpu.VMEM_SHARED((NBINS,), jnp.float32),
            pltpu.SemaphoreType.DMA,
            pltpu.SemaphoreType.DMA,
            pltpu.SemaphoreType.DMA,
        ],
    )


def _tc_body(p_ref, score_ref, ids_ref, val_ref):
    j = p_ref[...]
    rows = lax.broadcasted_iota(jnp.int32, (R, K), 0)
    cols = lax.broadcasted_iota(jnp.int32, (R, K), 1)
    jm = jnp.where(rows < C, j, 0.0)
    total = jnp.sum(jm, axis=0, keepdims=True)
    ctx = total - j
    tgt_tot = jnp.maximum(jnp.sum(j, axis=1, keepdims=True), 1.0)
    ctx_tot = jnp.maximum(jnp.sum(ctx, axis=1, keepdims=True), 1.0)
    score = jnp.log((j / tgt_tot + EPS) / (ctx / ctx_tot + EPS))
    neg_inf = jnp.float32(-jnp.inf)
    score_ref[...] = jnp.where(j[:C] >= 1.0, score[:C], neg_inf)
    work = jnp.where(j >= 1.0, score, -(10000.0 + cols.astype(jnp.float32)))
    kcols = lax.broadcasted_iota(jnp.int32, (R, NTOPK), 1)
    ids_acc = jnp.zeros((R, NTOPK), jnp.int32)
    val_acc = jnp.zeros((R, NTOPK), jnp.int32)
    jmin = jnp.full((R, 1), -1, jnp.int32)
    for step in range(NTOPK):
        work = jnp.where(cols == jmin, neg_inf, work)
        m = jnp.max(work, axis=1, keepdims=True)
        jmin = jnp.min(jnp.where(work == m, cols, K), axis=1, keepdims=True)
        ids_acc = jnp.where(kcols == step, jmin, ids_acc)
        val_acc = jnp.where(kcols == step,
                            (m > -9999.0).astype(jnp.int32), val_acc)
    ids_ref[...] = ids_acc[:C]
    val_ref[...] = val_acc[:C] != 0


_tc_select = pl.pallas_call(
    _tc_body,
    out_shape=(
        jax.ShapeDtypeStruct((C, K), jnp.float32),
        jax.ShapeDtypeStruct((C, NTOPK), jnp.int32),
        jax.ShapeDtypeStruct((C, NTOPK), jnp.bool_),
    ),
)


def kernel(indices, masks, num_classes, ignore_index):
    flat_idx = indices.reshape(-1).astype(jnp.int32)
    flat_msk = masks.reshape(-1).astype(jnp.int32)
    parts = _make_sc_hist()(flat_idx, flat_msk)
    score, ids, val = _tc_select(parts.reshape(NCORES * R, K))
    return ids, val, score


# speedup vs baseline: 1.1079x; 1.1079x over previous
import functools

import jax
import jax.numpy as jnp
from jax import lax
from jax.experimental import pallas as pl
from jax.experimental.pallas import tpu as pltpu
from jax.experimental.pallas import tpu_sc as plsc

K = 8192
C = 21
R = 24
NBINS = R * K
NTOPK = 64
EPS = 1e-6
NTOK = 64 * 32 * 32
NCORES = 2
NSUB = 16
NW = NCORES * NSUB
TPW = NTOK // NW
NCHUNK = TPW // 16
ZPW = NBINS // NSUB


def _sc_hist_body(idx_hbm, msk_hbm, out_hbm,
                  idx_v, msk_v, bins_v, ones_v, zeros_v, table_sh,
                  sem_ld, sem_z, sem_sc):
    c = lax.axis_index("c")
    s = lax.axis_index("s")
    wid = c * NSUB + s
    base = wid * TPW
    ld_idx = pltpu.async_copy(idx_hbm.at[pl.ds(base, TPW)], idx_v, sem_ld)
    ld_msk = pltpu.async_copy(msk_hbm.at[pl.ds(base, TPW)], msk_v, sem_ld)
    for i in range(8):
        ones_v[pl.ds(i * 16, 16)] = jnp.ones((16,), jnp.float32)
    for i in range(TPW // 16):
        zeros_v[pl.ds(i * 16, 16)] = jnp.zeros((16,), jnp.float32)
    zcopies = [
        pltpu.async_copy(zeros_v, table_sh.at[pl.ds(s * ZPW + i * TPW, TPW)],
                         sem_z)
        for i in range(ZPW // TPW)
    ]
    ld_idx.wait()
    ld_msk.wait()
    for t in range(NCHUNK):
        mi = msk_v[pl.ds(t * 16, 16)]
        ii = idx_v[pl.ds(t * 16, 16)]
        cls = jnp.minimum(mi, C)
        bins_v[t // 8, pl.ds((t % 8) * 16, 16)] = cls * K + ii
    for cp in zcopies:
        cp.wait()
    plsc.subcore_barrier()
    scopies = [
        pltpu.async_copy(ones_v, table_sh.at[bins_v.at[j]], sem_sc, add=True)
        for j in range(16)
    ]
    for cp in scopies:
        cp.wait()
    plsc.subcore_barrier()
    pltpu.sync_copy(table_sh.at[pl.ds(s * ZPW, ZPW)],
                    out_hbm.at[c, pl.ds(s * ZPW, ZPW)])


@functools.cache
def _make_sc_hist():
    return pl.kernel(
        _sc_hist_body,
        out_type=jax.ShapeDtypeStruct((NCORES, NBINS), jnp.float32),
        mesh=plsc.VectorSubcoreMesh(core_axis_name="c", subcore_axis_name="s",
                                    num_cores=NCORES, num_subcores=NSUB),
        scratch_types=[
            pltpu.VMEM((TPW,), jnp.int32),
            pltpu.VMEM((TPW,), jnp.int32),
            pltpu.VMEM((16, NCHUNK), jnp.int32),
            pltpu.VMEM((NCHUNK,), jnp.float32),
            pltpu.VMEM((TPW,), jnp.float32),
            pltpu.VMEM_SHARED((NBINS,), jnp.float32),
            pltpu.SemaphoreType.DMA,
            pltpu.SemaphoreType.DMA,
            pltpu.SemaphoreType.DMA,
        ],
    )


def _tc_body(p_ref, score_ref, ids_ref, val_ref):
    j = p_ref[:R] + p_ref[R:]
    rows = lax.broadcasted_iota(jnp.int32, (R, K), 0)
    cols = lax.broadcasted_iota(jnp.int32, (R, K), 1)
    jm = jnp.where(rows < C, j, 0.0)
    total = jnp.sum(jm, axis=0, keepdims=True)
    ctx = total - j
    tgt_tot = jnp.maximum(jnp.sum(j, axis=1, keepdims=True), 1.0)
    ctx_tot = jnp.maximum(jnp.sum(ctx, axis=1, keepdims=True), 1.0)
    score = jnp.log((j / tgt_tot + EPS) / (ctx / ctx_tot + EPS))
    neg_inf = jnp.float32(-jnp.inf)
    score_ref[...] = jnp.where(j[:C] >= 1.0, score[:C], neg_inf)
    work = jnp.where(j >= 1.0, score, -(10000.0 + cols.astype(jnp.float32)))
    kcols = lax.broadcasted_iota(jnp.int32, (R, NTOPK), 1)
    ids_acc = jnp.zeros((R, NTOPK), jnp.int32)
    val_acc = jnp.zeros((R, NTOPK), jnp.int32)
    j1 = jnp.full((R, 1), -1, jnp.int32)
    j2 = jnp.full((R, 1), -1, jnp.int32)
    for step in range(NTOPK // 2):
        work = jnp.where((cols == j1) | (cols == j2), neg_inf, work)
        m1 = jnp.max(work, axis=1, keepdims=True)
        eq1 = work == m1
        cand = jnp.where(eq1, cols, K)
        j1 = jnp.min(cand, axis=1, keepdims=True)
        jx = jnp.max(jnp.where(eq1, cols, -1), axis=1, keepdims=True)
        mx = jnp.max(jnp.where(eq1, neg_inf, work), axis=1, keepdims=True)
        m2 = jnp.where(jx != j1, m1, mx)
        j2 = jnp.min(jnp.where((work == m2) & (cols != j1), cols, K),
                     axis=1, keepdims=True)
        ids_acc = jnp.where(kcols == 2 * step, j1, ids_acc)
        ids_acc = jnp.where(kcols == 2 * step + 1, j2, ids_acc)
        val_acc = jnp.where(kcols == 2 * step,
                            (m1 > -9999.0).astype(jnp.int32), val_acc)
        val_acc = jnp.where(kcols == 2 * step + 1,
                            (m2 > -9999.0).astype(jnp.int32), val_acc)
    ids_ref[...] = ids_acc[:C]
    val_ref[...] = val_acc[:C] != 0


_tc_select = pl.pallas_call(
    _tc_body,
    out_shape=(
        jax.ShapeDtypeStruct((C, K), jnp.float32),
        jax.ShapeDtypeStruct((C, NTOPK), jnp.int32),
        jax.ShapeDtypeStruct((C, NTOPK), jnp.bool_),
    ),
)


def kernel(indices, masks, num_classes, ignore_index):
    flat_idx = indices.reshape(-1).astype(jnp.int32)
    flat_msk = masks.reshape(-1).astype(jnp.int32)
    parts = _make_sc_hist()(flat_idx, flat_msk)
    score, ids, val = _tc_select(parts.reshape(NCORES * R, K))
    return ids, val, score
